# baseline (speedup 1.0000x reference)
import functools

import jax
import jax.numpy as jnp
from jax import lax
from jax.experimental import pallas as pl
from jax.experimental.pallas import tpu as pltpu
from jax.experimental.pallas import tpu_sc as plsc

P, S, L, D, T = 8000, 8, 2000, 32, 16
HID, INNER = 32, 64
PB = 800
GRID_P = P // PB
W128 = 128

NC, NS = 2, 16
NW = NC * NS
_SC_MESH = plsc.VectorSubcoreMesh(core_axis_name="c", subcore_axis_name="s")


def _wid():
    return lax.axis_index("s") * NC + lax.axis_index("c")


def _sigmoid(x):
    return jax.nn.sigmoid(x)


def _gru_step(xz, h, U):
    H = h.shape[1]
    hz = jnp.dot(h, U[:, :2 * H], preferred_element_type=jnp.float32)
    z = _sigmoid(xz[:, :H] + hz[:, :H])
    r = _sigmoid(xz[:, H:2 * H] + hz[:, H:2 * H])
    hh = jnp.tanh(xz[:, 2 * H:] +
                  jnp.dot(r * h, U[:, 2 * H:], preferred_element_type=jnp.float32))
    return z * h + (1.0 - z) * hh


def _make_sc_gather(per_w, table_rows):
    n_full, tail = divmod(per_w, 128)
    chunks = [128] * n_full + ([tail] if tail else [])
    NB = 6

    @functools.partial(
        pl.kernel,
        out_type=jax.ShapeDtypeStruct((NW * per_w, W128), jnp.float32),
        mesh=_SC_MESH,
        scratch_types=(
            [pltpu.VMEM((per_w,), jnp.int32)]
            + [pltpu.VMEM((128, W128), jnp.float32) for _ in range(NB)]
            + [pltpu.SemaphoreType.DMA for _ in range(2 * NB)]
        ),
    )
    def gk(idx_hbm, table_hbm, out_hbm, idxv, *bufsems):
        bufs = bufsems[:NB]
        gsems = bufsems[NB:2 * NB]
        osems = bufsems[2 * NB:]
        base = _wid() * per_w
        pltpu.sync_copy(idx_hbm.at[pl.ds(base, per_w)], idxv)
        nch = len(chunks)
        LAG = NB - 1
        pend_g = {}
        pend_o = {}

        def writeback(j):
            b = j % NB
            pend_g.pop(j).wait()
            sz = chunks[j]
            pend_o[j] = pltpu.async_copy(
                bufs[b].at[pl.ds(0, sz)],
                out_hbm.at[pl.ds(base + j * 128, sz)], osems[b])

        for ci, sz in enumerate(chunks):
            b = ci % NB
            if ci - NB >= 0:
                pend_o.pop(ci - NB).wait()
            pend_g[ci] = pltpu.async_copy(
                table_hbm.at[idxv.at[pl.ds(ci * 128, sz)]],
                bufs[b].at[pl.ds(0, sz)], gsems[b])
            if ci >= LAG:
                writeback(ci - LAG)
        for j in range(max(0, nch - LAG), nch):
            writeback(j)
        for j in list(pend_o):
            pend_o.pop(j).wait()

    return gk


_GATHER_XZ = _make_sc_gather(P * S // NW, L)


GPW = 64


def _make_sc_segsum(table_rows, width):
    NB = 6

    @functools.partial(
        pl.kernel,
        out_type=jax.ShapeDtypeStruct((NW * GPW, width), jnp.float32),
        mesh=_SC_MESH,
        scratch_types=(
            [pltpu.VMEM((GPW * 32,), jnp.int32)]
            + [pltpu.VMEM((128,), jnp.int32) for _ in range(16)]
            + [pltpu.VMEM((128, width), jnp.float32) for _ in range(NB)]
            + [pltpu.VMEM((GPW, width), jnp.float32),
               pltpu.VMEM_SHARED((NS * GPW, width), jnp.float32)]
            + [pltpu.SemaphoreType.DMA for _ in range(2 * NB)]
        ),
    )
    def sk(idx_hbm, table_hbm, zeros_hbm, out_hbm, idxv, *rest):
        lidxs = rest[:16]
        bufs = rest[16:16 + NB]
        outbuf = rest[16 + NB]
        stage = rest[16 + NB + 1]
        gsems = rest[16 + NB + 2:16 + NB + 2 + NB]
        asems = rest[16 + NB + 2 + NB:]
        sid = lax.axis_index("s")
        base = _wid() * GPW
        sid_off = sid * GPW
        pltpu.sync_copy(idx_hbm.at[pl.ds(base * 32, GPW * 32)], idxv)
        pltpu.sync_copy(zeros_hbm, stage.at[pl.ds(sid_off, GPW)])
        for c in range(16):
            for kk in range(8):
                lidxs[c][pl.ds(kk * 16, 16)] = (
                    jnp.full((16,), c * 4 + kk // 2, jnp.int32) + sid_off)
        LAG = NB - 1
        pend_g = {}
        pend_a = {}

        def scatter(j):
            b = j % NB
            pend_g.pop(j).wait()
            pend_a[j] = pltpu.async_copy(
                bufs[b], stage.at[lidxs[j]], asems[b], add=True)

        for c in range(16):
            b = c % NB
            if c - NB >= 0:
                pend_a.pop(c - NB).wait()
            pend_g[c] = pltpu.async_copy(
                table_hbm.at[idxv.at[pl.ds(c * 128, 128)]],
                bufs[b], gsems[b])
            if c >= LAG:
                scatter(c - LAG)
        for j in range(16 - LAG, 16):
            scatter(j)
        for j in list(pend_a):
            pend_a.pop(j).wait()
        pltpu.sync_copy(stage.at[pl.ds(sid_off, GPW)], outbuf)
        pltpu.sync_copy(outbuf, out_hbm.at[pl.ds(base, GPW)])

    return sk


_SEGSUM_PSS = _make_sc_segsum(P * (S + 1), W128)
_SEGSUM_FT = _make_sc_segsum(P, W128)


EB = 2000
GRID_E = P // EB


def _enc_body(A_ref, B_ref, W0_ref, U0_ref, b0_ref, W1_ref, U1_ref, b1_ref,
              peW_ref, peb_ref, out_ref):
    h0s = [jnp.zeros((EB, INNER), jnp.float32) for _ in range(3)]
    h1s = [jnp.zeros((EB, HID), jnp.float32) for _ in range(3)]
    for t in range(T):
        for i in range(3):
            a_t = A_ref[i, :, t:t + 1]
            b_t = B_ref[i, :, t:t + 1]
            xz0 = (a_t * W0_ref[i, 0:1, :] + b_t * W0_ref[i, 1:2, :]
                   + b0_ref[i])
            h0s[i] = _gru_step(xz0, h0s[i], U0_ref[i])
            xz1 = (jnp.dot(h0s[i], W1_ref[i], preferred_element_type=jnp.float32)
                   + b1_ref[i])
            h1s[i] = _gru_step(xz1, h1s[i], U1_ref[i])
    enc = jnp.concatenate(h1s, axis=1)
    out_ref[...] = jax.nn.relu(
        jnp.dot(enc, peW_ref[...], preferred_element_type=jnp.float32)
        + peb_ref[...])


def _run_encoder(A, B, W0s, U0s, b0s, W1s, U1s, b1s, peW, peb):
    full = lambda shp: pl.BlockSpec(shp, lambda i: (0,) * len(shp))
    return pl.pallas_call(
        _enc_body,
        grid=(GRID_E,),
        in_specs=[
            pl.BlockSpec((3, EB, T), lambda i: (0, i, 0)),
            pl.BlockSpec((3, EB, T), lambda i: (0, i, 0)),
            full((3, 2, 3 * INNER)),
            full((3, INNER, 3 * INNER)),
            full((3, 1, 3 * INNER)),
            full((3, INNER, 3 * HID)),
            full((3, HID, 3 * HID)),
            full((3, 1, 3 * HID)),
            full((3 * HID, HID)),
            full((1, HID)),
        ],
        out_specs=pl.BlockSpec((EB, HID), lambda i: (i, 0)),
        out_shape=jax.ShapeDtypeStruct((P, HID), jnp.float32),
    )(A, B, W0s, U0s, b0s, W1s, U1s, b1s, peW, peb)


def _init_body(ls_ref, cap_ref, bt_ref,
               leW1_ref, leb1_ref, leW2_ref, leb2_ref,
               qeW1_ref, qeb1_ref, qeW2_ref, qeb2_ref,
               Wq_ref, Wl_ref, pub_ref,
               qs_ref, lsout_ref, qlw_ref):
    load = ls_ref[:, 0:1] / cap_ref[...]
    x = jax.nn.relu(load * leW1_ref[...] + leb1_ref[...])
    link_state = jax.nn.relu(
        jnp.dot(x, leW2_ref[...], preferred_element_type=jnp.float32) + leb2_ref[...])
    b0 = (bt_ref[...] == 0).astype(jnp.float32)
    x = jax.nn.relu(b0 * qeW1_ref[0:1, :] + (1.0 - b0) * qeW1_ref[1:2, :]
                    + qeb1_ref[...])
    queue_state = jax.nn.relu(
        jnp.dot(x, qeW2_ref[...], preferred_element_type=jnp.float32) + qeb2_ref[...])
    qs_ref[...] = queue_state
    lsout_ref[...] = link_state
    qlw = (jnp.dot(queue_state, Wq_ref[...], preferred_element_type=jnp.float32)
           + jnp.dot(link_state, Wl_ref[...], preferred_element_type=jnp.float32)
           + pub_ref[...])
    inv_cap = 1.0 / (cap_ref[...] * 1e9)
    qlw_ref[...] = jnp.concatenate(
        [qlw, inv_cap, jnp.zeros((L, W128 - 3 * HID - 1), jnp.float32)], axis=1)


def _run_init(ls, cap, bt, leW1, leb1, leW2, leb2, qeW1, qeb1, qeW2, qeb2,
              Wq, Wl, pub):
    return pl.pallas_call(
        _init_body,
        out_shape=(
            jax.ShapeDtypeStruct((L, HID), jnp.float32),
            jax.ShapeDtypeStruct((L, HID), jnp.float32),
            jax.ShapeDtypeStruct((L, W128), jnp.float32),
        ),
    )(ls, cap, bt, leW1, leb1, leW2, leb2, qeW1, qeb1, qeW2, qeb2, Wq, Wl, pub)


PGB = 2000
GRID_PG = P // PGB


def _pathgru_body(xz_ref, h_ref, U_ref, pss_ref):
    U = U_ref[...]
    h = h_ref[...]
    pss_ref[0, :, :HID] = h
    for s in range(S):
        h = _gru_step(xz_ref[:, s, :3 * HID], h, U)
        pss_ref[s + 1, :, :HID] = h


def _run_pathgru(xz_all, path_state, U):
    return pl.pallas_call(
        _pathgru_body,
        grid=(GRID_PG,),
        in_specs=[
            pl.BlockSpec((PGB, S, W128), lambda i: (i, 0, 0)),
            pl.BlockSpec((PGB, HID), lambda i: (i, 0)),
            pl.BlockSpec((HID, 3 * HID), lambda i: (0, 0)),
        ],
        out_specs=pl.BlockSpec((S + 1, PGB, W128), lambda i: (0, i, 0)),
        out_shape=jax.ShapeDtypeStruct((S + 1, P, W128), jnp.float32),
    )(xz_all, path_state, U)


def _qlup_body(pg_ref, qs_ref, ls_ref, icap_ref,
               quW_ref, quU_ref, qub_ref, luW_ref, luU_ref, lub_ref,
               Wq_ref, Wl_ref, pub_ref,
               qs2_ref, ls2_ref, qlw_ref):
    path_sum = pg_ref[:, :HID]
    qs = qs_ref[...]
    ls = ls_ref[...]
    qxz = jnp.dot(path_sum, quW_ref[...], preferred_element_type=jnp.float32) + qub_ref[...]
    qs2 = _gru_step(qxz, qs, quU_ref[...])
    lxz = jnp.dot(qs2, luW_ref[...], preferred_element_type=jnp.float32) + lub_ref[...]
    ls2 = _gru_step(lxz, ls, luU_ref[...])
    qs2_ref[...] = qs2
    ls2_ref[...] = ls2
    qlw = (jnp.dot(qs2, Wq_ref[...], preferred_element_type=jnp.float32)
           + jnp.dot(ls2, Wl_ref[...], preferred_element_type=jnp.float32)
           + pub_ref[...])
    qlw_ref[...] = jnp.concatenate(
        [qlw, icap_ref[...], jnp.zeros((L, W128 - 3 * HID - 1), jnp.float32)],
        axis=1)


def _run_qlup(path_gather, qs, ls, icap, quW, quU, qub, luW, luU, lub,
              Wq, Wl, pub):
    return pl.pallas_call(
        _qlup_body,
        out_shape=(
            jax.ShapeDtypeStruct((L, HID), jnp.float32),
            jax.ShapeDtypeStruct((L, HID), jnp.float32),
            jax.ShapeDtypeStruct((L, W128), jnp.float32),
        ),
    )(path_gather, qs, ls, icap, quW, quU, qub, luW, luU, lub, Wq, Wl, pub)


def _readout_body(pss_ref, cg_ref, len_ref, ft_ref, fp_ref,
                  W1_ref, b1_ref, W2_ref, b2_ref, W3_ref, b3_ref, out_ref):
    qd = jnp.zeros((PB, 1), jnp.float32)
    csum = jnp.zeros((PB, 1), jnp.float32)
    length = len_ref[...]
    for s in range(S):
        h = pss_ref[s + 1, :, :HID]
        o = jax.nn.relu(jnp.dot(h, W1_ref[...], preferred_element_type=jnp.float32) + b1_ref[...])
        o = jax.nn.relu(jnp.dot(o, W2_ref[...], preferred_element_type=jnp.float32) + b2_ref[...])
        occ = jnp.dot(o, W3_ref[...], preferred_element_type=jnp.float32) + b3_ref[...]
        m = (length > s).astype(jnp.float32)
        cgs = cg_ref[:, s:s + 1] * m
        qd = qd + occ * cgs
        csum = csum + cgs
    out_ref[...] = qd + (ft_ref[...] / fp_ref[...]) * csum


def _run_readout(pss, cg, length, ft, fp, W1, b1, W2, b2, W3, b3):
    full = lambda shp: pl.BlockSpec(shp, lambda i: (0,) * len(shp))
    return pl.pallas_call(
        _readout_body,
        grid=(GRID_P,),
        in_specs=[
            pl.BlockSpec((S + 1, PB, W128), lambda i: (0, i, 0)),
            pl.BlockSpec((PB, S), lambda i: (i, 0)),
            pl.BlockSpec((PB, 1), lambda i: (i, 0)),
            pl.BlockSpec((PB, 1), lambda i: (i, 0)),
            pl.BlockSpec((PB, 1), lambda i: (i, 0)),
            full((HID, 16)), full((1, 16)),
            full((16, 16)), full((1, 16)),
            full((16, 1)), full((1, 1)),
        ],
        out_specs=pl.BlockSpec((PB, 1), lambda i: (i, 0)),
        out_shape=jax.ShapeDtypeStruct((P, 1), jnp.float32),
    )(pss, cg, length, ft, fp, W1, b1, W2, b2, W3, b3)


def kernel(flow_traffic, flow_packets, flow_length, link_capacity, buffer_type,
           link_to_path, path_to_link, queue_to_link,
           flow_ipg_wt_cA, flow_packet_size_wt_cA,
           flow_ipg_wt_cD1, flow_packet_size_wt_cD1,
           flow_ipg_wt_cD2, flow_packet_size_wt_cD2, params):
    pr = params
    ft = flow_traffic
    p_idx = path_to_link[:, :, 0]
    p_pos = path_to_link[:, :, 1]
    ltp = link_to_path

    A = jnp.stack([flow_ipg_wt_cA[:, :, 0], flow_ipg_wt_cD1[:, :, 0],
                   flow_ipg_wt_cD2[:, :, 0]])
    B = jnp.stack([flow_packet_size_wt_cA[:, :, 0], flow_packet_size_wt_cD1[:, :, 0],
                   flow_packet_size_wt_cD2[:, :, 0]])
    W0s = jnp.stack([pr["rnn%d_l0" % i]["W"] for i in range(3)])
    U0s = jnp.stack([pr["rnn%d_l0" % i]["U"] for i in range(3)])
    b0s = jnp.stack([pr["rnn%d_l0" % i]["b"][None, :] for i in range(3)])
    W1s = jnp.stack([pr["rnn%d_l1" % i]["W"] for i in range(3)])
    U1s = jnp.stack([pr["rnn%d_l1" % i]["U"] for i in range(3)])
    b1s = jnp.stack([pr["rnn%d_l1" % i]["b"][None, :] for i in range(3)])

    path_state = _run_encoder(A, B, W0s, U0s, b0s, W1s, U1s, b1s,
                              pr["pe_W"], pr["pe_b"][None, :])

    zeros64 = jnp.zeros((GPW, W128), jnp.float32)
    pidx_flat = jnp.pad(p_idx.reshape(-1), (0, NW * GPW * 32 - L * D))
    ft_pad = jnp.pad(ft, ((0, 0), (0, W128 - 1)))
    ls = _SEGSUM_FT(pidx_flat, ft_pad, zeros64)[:L]
    pu = pr["pu"]
    Wq = pu["W"][:HID, :]
    Wl = pu["W"][HID:, :]
    pub = pu["b"][None, :]
    queue_state, link_state, qlw = _run_init(
        ls, link_capacity, buffer_type,
        pr["le_W1"], pr["le_b1"][None, :], pr["le_W2"], pr["le_b2"][None, :],
        pr["qe_W1"], pr["qe_b1"][None, :], pr["qe_W2"], pr["qe_b2"][None, :],
        Wq, Wl, pub)

    qu, lu = pr["qu"], pr["lu"]
    idx2 = jnp.pad((p_pos * P + p_idx).reshape(-1),
                   (0, NW * GPW * 32 - L * D))
    ltp_flat = ltp.reshape(-1)

    def mp_iter(_, carry):
        qlw, queue_state, link_state, path_state, _pss, _capg = carry
        xz_all = _GATHER_XZ(ltp_flat, qlw).reshape(P, S, W128)
        capg = xz_all[:, :, 3 * HID]
        pss = _run_pathgru(xz_all, path_state, pu["U"])
        path_state = pss[S, :, :HID]
        flat = pss.reshape((S + 1) * P, W128)
        path_gather = _SEGSUM_PSS(idx2, flat, zeros64)[:L]
        icap = qlw[:, 3 * HID:3 * HID + 1]
        queue_state, link_state, qlw = _run_qlup(
            path_gather, queue_state, link_state, icap,
            qu["W"], qu["U"], qu["b"][None, :],
            lu["W"], lu["U"], lu["b"][None, :], Wq, Wl, pub)
        return (qlw, queue_state, link_state, path_state, pss, capg)

    pss0 = jnp.zeros((S + 1, P, W128), jnp.float32)
    capg0 = jnp.zeros((P, S), jnp.float32)
    carry = (qlw, queue_state, link_state, path_state, pss0, capg0)
    _, _, _, _, pss, capg = lax.fori_loop(0, 8, mp_iter, carry)

    return _run_readout(pss, capg, flow_length, ft, flow_packets,
                        pr["ro_W1"], pr["ro_b1"][None, :],
                        pr["ro_W2"], pr["ro_b2"][None, :],
                        pr["ro_W3"], pr["ro_b3"][None, :])


# speedup vs baseline: 1.1126x; 1.1126x over previous
import functools

import jax
import jax.numpy as jnp
from jax import lax
from jax.experimental import pallas as pl
from jax.experimental.pallas import tpu as pltpu
from jax.experimental.pallas import tpu_sc as plsc

P, S, L, D, T = 8000, 8, 2000, 32, 16
HID, INNER = 32, 64
PB = 800
GRID_P = P // PB
W128 = 128

NC, NS = 2, 16
NW = NC * NS
_SC_MESH = plsc.VectorSubcoreMesh(core_axis_name="c", subcore_axis_name="s")


def _wid():
    return lax.axis_index("s") * NC + lax.axis_index("c")


def _sigmoid(x):
    return jax.nn.sigmoid(x)


def _gru_step(xz, h, U):
    H = h.shape[1]
    hz = jnp.dot(h, U[:, :2 * H], preferred_element_type=jnp.float32)
    z = _sigmoid(xz[:, :H] + hz[:, :H])
    r = _sigmoid(xz[:, H:2 * H] + hz[:, H:2 * H])
    hh = jnp.tanh(xz[:, 2 * H:] +
                  jnp.dot(r * h, U[:, 2 * H:], preferred_element_type=jnp.float32))
    return z * h + (1.0 - z) * hh


def _make_sc_gather(per_w, table_rows):
    n_full, tail = divmod(per_w, 128)
    chunks = [128] * n_full + ([tail] if tail else [])
    NB = 6

    @functools.partial(
        pl.kernel,
        out_type=jax.ShapeDtypeStruct((NW * per_w, W128), jnp.float32),
        mesh=_SC_MESH,
        scratch_types=(
            [pltpu.VMEM((per_w,), jnp.int32)]
            + [pltpu.VMEM((128, W128), jnp.float32) for _ in range(NB)]
            + [pltpu.SemaphoreType.DMA for _ in range(2 * NB)]
        ),
    )
    def gk(idx_hbm, table_hbm, out_hbm, idxv, *bufsems):
        bufs = bufsems[:NB]
        gsems = bufsems[NB:2 * NB]
        osems = bufsems[2 * NB:]
        base = _wid() * per_w
        pltpu.sync_copy(idx_hbm.at[pl.ds(base, per_w)], idxv)
        nch = len(chunks)
        LAG = NB - 1
        pend_g = {}
        pend_o = {}

        def writeback(j):
            b = j % NB
            pend_g.pop(j).wait()
            sz = chunks[j]
            pend_o[j] = pltpu.async_copy(
                bufs[b].at[pl.ds(0, sz)],
                out_hbm.at[pl.ds(base + j * 128, sz)], osems[b])

        for ci, sz in enumerate(chunks):
            b = ci % NB
            if ci - NB >= 0:
                pend_o.pop(ci - NB).wait()
            pend_g[ci] = pltpu.async_copy(
                table_hbm.at[idxv.at[pl.ds(ci * 128, sz)]],
                bufs[b].at[pl.ds(0, sz)], gsems[b])
            if ci >= LAG:
                writeback(ci - LAG)
        for j in range(max(0, nch - LAG), nch):
            writeback(j)
        for j in list(pend_o):
            pend_o.pop(j).wait()

    return gk


_GATHER_XZ = _make_sc_gather(P * S // NW, L)


GPW = 64


def _make_sc_segsum(table_rows, width):
    NB = 6

    @functools.partial(
        pl.kernel,
        out_type=jax.ShapeDtypeStruct((NW * GPW, width), jnp.float32),
        mesh=_SC_MESH,
        scratch_types=(
            [pltpu.VMEM((GPW * 32,), jnp.int32)]
            + [pltpu.VMEM((128,), jnp.int32) for _ in range(16)]
            + [pltpu.VMEM((128, width), jnp.float32) for _ in range(NB)]
            + [pltpu.VMEM((GPW, width), jnp.float32),
               pltpu.VMEM_SHARED((NS * GPW, width), jnp.float32)]
            + [pltpu.SemaphoreType.DMA for _ in range(2 * NB)]
        ),
    )
    def sk(idx_hbm, table_hbm, zeros_hbm, out_hbm, idxv, *rest):
        lidxs = rest[:16]
        bufs = rest[16:16 + NB]
        outbuf = rest[16 + NB]
        stage = rest[16 + NB + 1]
        gsems = rest[16 + NB + 2:16 + NB + 2 + NB]
        asems = rest[16 + NB + 2 + NB:]
        sid = lax.axis_index("s")
        base = _wid() * GPW
        sid_off = sid * GPW
        pltpu.sync_copy(idx_hbm.at[pl.ds(base * 32, GPW * 32)], idxv)
        pltpu.sync_copy(zeros_hbm, stage.at[pl.ds(sid_off, GPW)])
        for c in range(16):
            for kk in range(8):
                lidxs[c][pl.ds(kk * 16, 16)] = (
                    jnp.full((16,), c * 4 + kk // 2, jnp.int32) + sid_off)
        LAG = NB - 1
        pend_g = {}
        pend_a = {}

        def scatter(j):
            b = j % NB
            pend_g.pop(j).wait()
            pend_a[j] = pltpu.async_copy(
                bufs[b], stage.at[lidxs[j]], asems[b], add=True)

        for c in range(16):
            b = c % NB
            if c - NB >= 0:
                pend_a.pop(c - NB).wait()
            pend_g[c] = pltpu.async_copy(
                table_hbm.at[idxv.at[pl.ds(c * 128, 128)]],
                bufs[b], gsems[b])
            if c >= LAG:
                scatter(c - LAG)
        for j in range(16 - LAG, 16):
            scatter(j)
        for j in list(pend_a):
            pend_a.pop(j).wait()
        pltpu.sync_copy(stage.at[pl.ds(sid_off, GPW)], outbuf)
        pltpu.sync_copy(outbuf, out_hbm.at[pl.ds(base, GPW)])

    return sk


_SEGSUM_PSS = _make_sc_segsum(P * (S + 1), W128)
_SEGSUM_FT = _make_sc_segsum(P, W128)


EB = 2000
GRID_E = P // EB


def _enc_body(A_ref, B_ref, W0_ref, U0_ref, b0_ref, W1_ref, U1_ref, b1_ref,
              peW_ref, peb_ref, out_ref):
    h0s = [jnp.zeros((EB, INNER), jnp.float32) for _ in range(3)]
    h1s = [jnp.zeros((EB, HID), jnp.float32) for _ in range(3)]
    for t in range(T):
        for i in range(3):
            a_t = A_ref[i, :, t:t + 1]
            b_t = B_ref[i, :, t:t + 1]
            xz0 = (a_t * W0_ref[i, 0:1, :] + b_t * W0_ref[i, 1:2, :]
                   + b0_ref[i])
            h0s[i] = _gru_step(xz0, h0s[i], U0_ref[i])
            xz1 = (jnp.dot(h0s[i], W1_ref[i], preferred_element_type=jnp.float32)
                   + b1_ref[i])
            h1s[i] = _gru_step(xz1, h1s[i], U1_ref[i])
    enc = jnp.concatenate(h1s, axis=1)
    out_ref[...] = jax.nn.relu(
        jnp.dot(enc, peW_ref[...], preferred_element_type=jnp.float32)
        + peb_ref[...])


def _run_encoder(A, B, W0s, U0s, b0s, W1s, U1s, b1s, peW, peb):
    full = lambda shp: pl.BlockSpec(shp, lambda i: (0,) * len(shp))
    return pl.pallas_call(
        _enc_body,
        grid=(GRID_E,),
        in_specs=[
            pl.BlockSpec((3, EB, T), lambda i: (0, i, 0)),
            pl.BlockSpec((3, EB, T), lambda i: (0, i, 0)),
            full((3, 2, 3 * INNER)),
            full((3, INNER, 3 * INNER)),
            full((3, 1, 3 * INNER)),
            full((3, INNER, 3 * HID)),
            full((3, HID, 3 * HID)),
            full((3, 1, 3 * HID)),
            full((3 * HID, HID)),
            full((1, HID)),
        ],
        out_specs=pl.BlockSpec((EB, HID), lambda i: (i, 0)),
        out_shape=jax.ShapeDtypeStruct((P, HID), jnp.float32),
    )(A, B, W0s, U0s, b0s, W1s, U1s, b1s, peW, peb)


def _init_body(ls_ref, cap_ref, bt_ref,
               leW1_ref, leb1_ref, leW2_ref, leb2_ref,
               qeW1_ref, qeb1_ref, qeW2_ref, qeb2_ref,
               Wq_ref, Wl_ref, pub_ref,
               qs_ref, lsout_ref, qlw_ref):
    load = ls_ref[:, 0:1] / cap_ref[...]
    x = jax.nn.relu(load * leW1_ref[...] + leb1_ref[...])
    link_state = jax.nn.relu(
        jnp.dot(x, leW2_ref[...], preferred_element_type=jnp.float32) + leb2_ref[...])
    b0 = (bt_ref[...] == 0).astype(jnp.float32)
    x = jax.nn.relu(b0 * qeW1_ref[0:1, :] + (1.0 - b0) * qeW1_ref[1:2, :]
                    + qeb1_ref[...])
    queue_state = jax.nn.relu(
        jnp.dot(x, qeW2_ref[...], preferred_element_type=jnp.float32) + qeb2_ref[...])
    qs_ref[...] = queue_state
    lsout_ref[...] = link_state
    qlw = (jnp.dot(queue_state, Wq_ref[...], preferred_element_type=jnp.float32)
           + jnp.dot(link_state, Wl_ref[...], preferred_element_type=jnp.float32)
           + pub_ref[...])
    inv_cap = 1.0 / (cap_ref[...] * 1e9)
    qlw_ref[...] = jnp.concatenate(
        [qlw, inv_cap, jnp.zeros((L, W128 - 3 * HID - 1), jnp.float32)], axis=1)


def _run_init(ls, cap, bt, leW1, leb1, leW2, leb2, qeW1, qeb1, qeW2, qeb2,
              Wq, Wl, pub):
    return pl.pallas_call(
        _init_body,
        out_shape=(
            jax.ShapeDtypeStruct((L, HID), jnp.float32),
            jax.ShapeDtypeStruct((L, HID), jnp.float32),
            jax.ShapeDtypeStruct((L, W128), jnp.float32),
        ),
    )(ls, cap, bt, leW1, leb1, leW2, leb2, qeW1, qeb1, qeW2, qeb2, Wq, Wl, pub)


PGB = 1000
GRID_PG = P // PGB


def _pathgru_body(xz_ref, h_ref, U_ref, pss_ref):
    U = U_ref[...]
    h = h_ref[...]
    pss_ref[0, :, :HID] = h
    for s in range(S):
        h = _gru_step(xz_ref[:, s, :3 * HID], h, U)
        pss_ref[s + 1, :, :HID] = h


def _run_pathgru(xz_all, path_state, U):
    return pl.pallas_call(
        _pathgru_body,
        grid=(GRID_PG,),
        in_specs=[
            pl.BlockSpec((PGB, S, W128), lambda i: (i, 0, 0)),
            pl.BlockSpec((PGB, HID), lambda i: (i, 0)),
            pl.BlockSpec((HID, 3 * HID), lambda i: (0, 0)),
        ],
        out_specs=pl.BlockSpec((S + 1, PGB, W128), lambda i: (0, i, 0)),
        out_shape=jax.ShapeDtypeStruct((S + 1, P, W128), jnp.float32),
    )(xz_all, path_state, U)


def _qlup_body(pg_ref, qs_ref, ls_ref, icap_ref,
               quW_ref, quU_ref, qub_ref, luW_ref, luU_ref, lub_ref,
               Wq_ref, Wl_ref, pub_ref,
               qs2_ref, ls2_ref, qlw_ref):
    path_sum = pg_ref[:, :HID]
    qs = qs_ref[...]
    ls = ls_ref[...]
    qxz = jnp.dot(path_sum, quW_ref[...], preferred_element_type=jnp.float32) + qub_ref[...]
    qs2 = _gru_step(qxz, qs, quU_ref[...])
    lxz = jnp.dot(qs2, luW_ref[...], preferred_element_type=jnp.float32) + lub_ref[...]
    ls2 = _gru_step(lxz, ls, luU_ref[...])
    qs2_ref[...] = qs2
    ls2_ref[...] = ls2
    qlw = (jnp.dot(qs2, Wq_ref[...], preferred_element_type=jnp.float32)
           + jnp.dot(ls2, Wl_ref[...], preferred_element_type=jnp.float32)
           + pub_ref[...])
    qlw_ref[...] = jnp.concatenate(
        [qlw, icap_ref[...], jnp.zeros((L, W128 - 3 * HID - 1), jnp.float32)],
        axis=1)


def _run_qlup(path_gather, qs, ls, icap, quW, quU, qub, luW, luU, lub,
              Wq, Wl, pub):
    return pl.pallas_call(
        _qlup_body,
        out_shape=(
            jax.ShapeDtypeStruct((L, HID), jnp.float32),
            jax.ShapeDtypeStruct((L, HID), jnp.float32),
            jax.ShapeDtypeStruct((L, W128), jnp.float32),
        ),
    )(path_gather, qs, ls, icap, quW, quU, qub, luW, luU, lub, Wq, Wl, pub)


def _readout_body(pss_ref, cg_ref, len_ref, ft_ref, fp_ref,
                  W1_ref, b1_ref, W2_ref, b2_ref, W3_ref, b3_ref, out_ref):
    qd = jnp.zeros((PB, 1), jnp.float32)
    csum = jnp.zeros((PB, 1), jnp.float32)
    length = len_ref[...]
    for s in range(S):
        h = pss_ref[s + 1, :, :HID]
        o = jax.nn.relu(jnp.dot(h, W1_ref[...], preferred_element_type=jnp.float32) + b1_ref[...])
        o = jax.nn.relu(jnp.dot(o, W2_ref[...], preferred_element_type=jnp.float32) + b2_ref[...])
        occ = jnp.dot(o, W3_ref[...], preferred_element_type=jnp.float32) + b3_ref[...]
        m = (length > s).astype(jnp.float32)
        cgs = cg_ref[:, s:s + 1] * m
        qd = qd + occ * cgs
        csum = csum + cgs
    out_ref[...] = qd + (ft_ref[...] / fp_ref[...]) * csum


def _run_readout(pss, cg, length, ft, fp, W1, b1, W2, b2, W3, b3):
    full = lambda shp: pl.BlockSpec(shp, lambda i: (0,) * len(shp))
    return pl.pallas_call(
        _readout_body,
        grid=(GRID_P,),
        in_specs=[
            pl.BlockSpec((S + 1, PB, W128), lambda i: (0, i, 0)),
            pl.BlockSpec((PB, S), lambda i: (i, 0)),
            pl.BlockSpec((PB, 1), lambda i: (i, 0)),
            pl.BlockSpec((PB, 1), lambda i: (i, 0)),
            pl.BlockSpec((PB, 1), lambda i: (i, 0)),
            full((HID, 16)), full((1, 16)),
            full((16, 16)), full((1, 16)),
            full((16, 1)), full((1, 1)),
        ],
        out_specs=pl.BlockSpec((PB, 1), lambda i: (i, 0)),
        out_shape=jax.ShapeDtypeStruct((P, 1), jnp.float32),
    )(pss, cg, length, ft, fp, W1, b1, W2, b2, W3, b3)


def kernel(flow_traffic, flow_packets, flow_length, link_capacity, buffer_type,
           link_to_path, path_to_link, queue_to_link,
           flow_ipg_wt_cA, flow_packet_size_wt_cA,
           flow_ipg_wt_cD1, flow_packet_size_wt_cD1,
           flow_ipg_wt_cD2, flow_packet_size_wt_cD2, params):
    pr = params
    ft = flow_traffic
    p_idx = path_to_link[:, :, 0]
    p_pos = path_to_link[:, :, 1]
    ltp = link_to_path

    A = jnp.stack([flow_ipg_wt_cA[:, :, 0], flow_ipg_wt_cD1[:, :, 0],
                   flow_ipg_wt_cD2[:, :, 0]])
    B = jnp.stack([flow_packet_size_wt_cA[:, :, 0], flow_packet_size_wt_cD1[:, :, 0],
                   flow_packet_size_wt_cD2[:, :, 0]])
    W0s = jnp.stack([pr["rnn%d_l0" % i]["W"] for i in range(3)])
    U0s = jnp.stack([pr["rnn%d_l0" % i]["U"] for i in range(3)])
    b0s = jnp.stack([pr["rnn%d_l0" % i]["b"][None, :] for i in range(3)])
    W1s = jnp.stack([pr["rnn%d_l1" % i]["W"] for i in range(3)])
    U1s = jnp.stack([pr["rnn%d_l1" % i]["U"] for i in range(3)])
    b1s = jnp.stack([pr["rnn%d_l1" % i]["b"][None, :] for i in range(3)])

    path_state = _run_encoder(A, B, W0s, U0s, b0s, W1s, U1s, b1s,
                              pr["pe_W"], pr["pe_b"][None, :])

    zeros64 = jnp.zeros((GPW, W128), jnp.float32)
    pidx_flat = jnp.pad(p_idx.reshape(-1), (0, NW * GPW * 32 - L * D))
    ft_pad = jnp.pad(ft, ((0, 0), (0, W128 - 1)))
    ls = _SEGSUM_FT(pidx_flat, ft_pad, zeros64)[:L]
    pu = pr["pu"]
    Wq = pu["W"][:HID, :]
    Wl = pu["W"][HID:, :]
    pub = pu["b"][None, :]
    queue_state, link_state, qlw = _run_init(
        ls, link_capacity, buffer_type,
        pr["le_W1"], pr["le_b1"][None, :], pr["le_W2"], pr["le_b2"][None, :],
        pr["qe_W1"], pr["qe_b1"][None, :], pr["qe_W2"], pr["qe_b2"][None, :],
        Wq, Wl, pub)

    qu, lu = pr["qu"], pr["lu"]
    idx2 = jnp.pad((p_pos * P + p_idx).reshape(-1),
                   (0, NW * GPW * 32 - L * D))
    ltp_flat = ltp.reshape(-1)

    def mp_iter(_, carry):
        qlw, queue_state, link_state, path_state, _pss, _capg = carry
        xz_all = _GATHER_XZ(ltp_flat, qlw).reshape(P, S, W128)
        capg = xz_all[:, :, 3 * HID]
        pss = _run_pathgru(xz_all, path_state, pu["U"])
        path_state = pss[S, :, :HID]
        flat = pss.reshape((S + 1) * P, W128)
        path_gather = _SEGSUM_PSS(idx2, flat, zeros64)[:L]
        icap = qlw[:, 3 * HID:3 * HID + 1]
        queue_state, link_state, qlw = _run_qlup(
            path_gather, queue_state, link_state, icap,
            qu["W"], qu["U"], qu["b"][None, :],
            lu["W"], lu["U"], lu["b"][None, :], Wq, Wl, pub)
        return (qlw, queue_state, link_state, path_state, pss, capg)

    pss0 = jnp.zeros((S + 1, P, W128), jnp.float32)
    capg0 = jnp.zeros((P, S), jnp.float32)
    carry = (qlw, queue_state, link_state, path_state, pss0, capg0)
    _, _, _, _, pss, capg = lax.fori_loop(0, 8, mp_iter, carry)

    return _run_readout(pss, capg, flow_length, ft, flow_packets,
                        pr["ro_W1"], pr["ro_b1"][None, :],
                        pr["ro_W2"], pr["ro_b2"][None, :],
                        pr["ro_W3"], pr["ro_b3"][None, :])
